# baseline (speedup 1.0000x reference)
import jax
import jax.numpy as jnp
from jax import lax
from jax.experimental import pallas as pl
from jax.experimental.pallas import tpu as pltpu
from jax.experimental.pallas import tpu_sc as plsc

_TOTAL = 32768
_NSEG = 16
_L = 16
_WINDOW = 2064
_WIN_ALIGN_MAX = _TOTAL - _WINDOW
_NCORES = 2
_NSUB = 16
_CHUNK = _TOTAL // (_NCORES * _NSUB)
_NVEC = _CHUNK // _L
_NEG_INF = float('-inf')


def _sc_body(scores_hbm, lens_hbm, out_hbm,
             lens_v, win_v, chunk_v, outc_v, stats_v, allstats_v, shared):
    cid = lax.axis_index("c")
    sid = lax.axis_index("s")
    wid = cid * _NSUB + sid
    lane = lax.iota(jnp.int32, _L)

    pltpu.sync_copy(lens_hbm, lens_v)
    lens = lens_v[...]
    ends = plsc.cumsum(lens)

    def exi(vec, i):
        return jnp.max(jnp.where(lane == i, vec, 0))

    def exf(vec, i, fill):
        return jnp.max(jnp.where(lane == i, vec, fill))

    seg_end = exi(ends, sid)
    seg_len = exi(lens, sid)
    seg_start = seg_end - seg_len
    total = jnp.max(ends)

    base = wid * _CHUNK
    pltpu.sync_copy(scores_hbm.at[pl.ds(base, _CHUNK)], chunk_v)

    astart = jnp.minimum((seg_start // 8) * 8, _WIN_ALIGN_MAX)
    pltpu.sync_copy(scores_hbm.at[pl.ds(astart, _WINDOW)], win_v)
    off = seg_start - astart
    j_lo = off // _L
    j_hi = jnp.where(seg_len > 0, (off + seg_len + _L - 1) // _L, j_lo)
    jf_lo = (off + _L - 1) // _L
    jf_hi = jnp.maximum((off + seg_len) // _L, jf_lo)
    head_hi = jnp.minimum(jf_lo, j_hi)
    tail_lo = jnp.minimum(jnp.maximum(jf_hi, head_hi), j_hi)

    def body_max_masked(j, acc):
        x = win_v[pl.ds(j * _L, _L)]
        pos = astart + j * _L + lane
        m = (pos >= seg_start) & (pos < seg_end)
        return jnp.maximum(acc, jnp.where(m, x, _NEG_INF))

    def body_max(j, acc):
        return jnp.maximum(acc, win_v[pl.ds(j * _L, _L)])

    maxacc = jnp.full((_L,), _NEG_INF, jnp.float32)
    maxacc = lax.fori_loop(j_lo, head_hi, body_max_masked, maxacc)
    maxacc = lax.fori_loop(jf_lo, jf_hi, body_max, maxacc)
    maxacc = lax.fori_loop(tail_lo, j_hi, body_max_masked, maxacc)
    mval = jnp.max(maxacc)

    def body_sum_masked(j, acc):
        x = win_v[pl.ds(j * _L, _L)]
        pos = astart + j * _L + lane
        m = (pos >= seg_start) & (pos < seg_end)
        return acc + jnp.where(m, jnp.exp(x - mval), 0.0)

    def body_sum(j, acc):
        return acc + jnp.exp(win_v[pl.ds(j * _L, _L)] - mval)

    sumacc = jnp.zeros((_L,), jnp.float32)
    sumacc = lax.fori_loop(j_lo, head_hi, body_sum_masked, sumacc)
    sumacc = lax.fori_loop(jf_lo, jf_hi, body_sum, sumacc)
    sumacc = lax.fori_loop(tail_lo, j_hi, body_sum_masked, sumacc)
    denom = jnp.sum(sumacc)
    invd_vec = (jnp.full((_L,), 1.0, jnp.float32)
                / jnp.full((_L,), denom, jnp.float32))

    stats_v[0, :] = jnp.where(lane == sid, mval, _NEG_INF)
    stats_v[1, :] = jnp.where(lane == sid, invd_vec, 0.0)
    pltpu.sync_copy(stats_v, shared.at[sid])
    plsc.subcore_barrier()
    pltpu.sync_copy(shared, allstats_v)

    maxs = allstats_v[0, 0, :]
    invds = allstats_v[0, 1, :]
    for r in range(1, _NSEG):
        maxs = jnp.maximum(maxs, allstats_v[r, 0, :])
        invds = invds + allstats_v[r, 1, :]

    j_init = jnp.minimum(jnp.maximum(total - base, 0) // _L, _NVEC)

    def body_init(j, carry):
        outc_v[pl.ds(j * _L, _L)] = chunk_v[pl.ds(j * _L, _L)]
        return carry

    lax.fori_loop(j_init, _NVEC, body_init, 0)

    prev_end = jnp.int32(0)
    for s in range(_NSEG):
        e_s = exi(ends, s)
        st_s = prev_end
        prev_end = e_s
        m_s = exf(maxs, s, _NEG_INF)
        d_s = exf(invds, s, 0.0)
        lo = jnp.clip(st_s - base, 0, _CHUNK)
        hi = jnp.clip(e_s - base, 0, _CHUNK)
        jlo = lo // _L
        jhi = jnp.where(hi > lo, (hi + _L - 1) // _L, jlo)
        jflo = (lo + _L - 1) // _L
        jfhi = jnp.maximum(hi // _L, jflo)
        h_hi = jnp.minimum(jflo, jhi)
        t_lo = jnp.minimum(jnp.maximum(jfhi, h_hi), jhi)

        def body_seg_masked(j, carry, st_s=st_s, e_s=e_s, m_s=m_s, d_s=d_s):
            x = chunk_v[pl.ds(j * _L, _L)]
            pos = base + j * _L + lane
            msk = (pos >= st_s) & (pos < e_s)
            soft = jnp.exp(x - m_s) * d_s
            prev = outc_v[pl.ds(j * _L, _L)]
            outc_v[pl.ds(j * _L, _L)] = jnp.where(msk, soft, prev)
            return carry

        def body_seg(j, carry, m_s=m_s, d_s=d_s):
            x = chunk_v[pl.ds(j * _L, _L)]
            outc_v[pl.ds(j * _L, _L)] = jnp.exp(x - m_s) * d_s
            return carry

        lax.fori_loop(jlo, jhi, body_seg_masked, 0)

    pltpu.sync_copy(outc_v, out_hbm.at[pl.ds(base, _CHUNK)])


def _make_kernel(interpret=False):
    mesh = plsc.VectorSubcoreMesh(core_axis_name="c", subcore_axis_name="s",
                                  num_cores=_NCORES, num_subcores=_NSUB)
    return pl.kernel(
        _sc_body,
        out_type=jax.ShapeDtypeStruct((_TOTAL,), jnp.float32),
        mesh=mesh,
        scratch_types=[
            pltpu.VMEM((_NSEG,), jnp.int32),
            pltpu.VMEM((_WINDOW,), jnp.float32),
            pltpu.VMEM((_CHUNK,), jnp.float32),
            pltpu.VMEM((_CHUNK,), jnp.float32),
            pltpu.VMEM((2, _L), jnp.float32),
            pltpu.VMEM((_NSEG, 2, _L), jnp.float32),
            pltpu.VMEM_SHARED((_NSEG, 2, _L), jnp.float32),
        ],
        compiler_params=pltpu.CompilerParams(needs_layout_passes=False),
        interpret=interpret,
    )


_sc_kernel = _make_kernel()


def kernel(scores, nBestIndex):
    return _sc_kernel(scores, nBestIndex.astype(jnp.int32))
